# Initial kernel scaffold; baseline (speedup 1.0000x reference)
import jax
import jax.numpy as jnp
from jax.experimental import pallas as pl


def kernel(x, fc1_w, fc1_b, fc2_w, fc2_b, cls1_w, cls1_b, cls2_w, cls2_b):
    raise NotImplementedError("write your pallas kernel here")


# speedup vs baseline: 6.0399x; 6.0399x over previous
import jax
import jax.numpy as jnp
from jax.experimental import pallas as pl
from jax.experimental.pallas import tpu as pltpu

_EPS_W = 1e-6
_EPS_P = 1e-12

_CH = 8


def _stats_body(x_ref, sx_ref, s0_ref, s1_ref, t0_ref, t1_ref):
    x = x_ref[...]
    sl = jnp.concatenate([x[:, :, 1:], x[:, :, -1:]], axis=2)
    sr = jnp.concatenate([x[:, :, :1], x[:, :, :-1]], axis=2)
    rmax = jnp.maximum(jnp.maximum(x, sl), sr)
    rmin = jnp.minimum(jnp.minimum(x, sl), sr)
    up = jnp.concatenate([rmax[:, 1:, :], rmax[:, -1:, :]], axis=1)
    dn = jnp.concatenate([rmax[:, :1, :], rmax[:, :-1, :]], axis=1)
    mx = jnp.maximum(jnp.maximum(rmax, up), dn)
    up = jnp.concatenate([rmin[:, 1:, :], rmin[:, -1:, :]], axis=1)
    dn = jnp.concatenate([rmin[:, :1, :], rmin[:, :-1, :]], axis=1)
    mn = jnp.minimum(jnp.minimum(rmin, up), dn)
    l0 = mx - x
    l1 = x - mn
    g0 = l0 * jnp.log(l0 + _EPS_P)
    g1 = l1 * jnp.log(l1 + _EPS_P)

    def red(v):
        return jnp.sum(v, axis=(1, 2)).reshape(1, 1, _CH)

    sx_ref[...] = red(x)
    s0_ref[...] = red(l0)
    s1_ref[...] = red(l1)
    t0_ref[...] = red(g0)
    t1_ref[...] = red(g1)


def _head_body(sx_ref, s0_ref, s1_ref, t0_ref, t1_ref,
               fc1_wt_ref, fc1_b_ref, fc2_wt_ref, fc2_b_ref,
               cls1_wt_ref, cls1_b_ref, cls2_wt_ref, cls2_b_ref,
               inv_hw_ref, logits_ref, w_ref, f_ref):
    z = sx_ref[...] * inv_hw_ref[0, 0]
    h = jnp.dot(z, fc1_wt_ref[...], preferred_element_type=jnp.float32)
    h = jnp.maximum(h + fc1_b_ref[...], 0.0)
    a = jnp.dot(h, fc2_wt_ref[...], preferred_element_type=jnp.float32)
    a = a + fc2_b_ref[...]
    a = a - jnp.max(a, axis=1, keepdims=True)
    e = jnp.exp(a)
    w = e / jnp.sum(e, axis=1, keepdims=True)

    def entropy(s_ref, t_ref):
        s = s_ref[...]
        t = t_ref[...]
        sp = s + _EPS_P
        return jnp.where(s > 0, jnp.log(sp) - t / sp, 0.0)

    ent0 = entropy(s0_ref, t0_ref)
    ent1 = entropy(s1_ref, t1_ref)
    wn = w / (jnp.sum(w, axis=1, keepdims=True) + _EPS_W)
    f0 = jnp.sum(ent0 * wn, axis=1, keepdims=True)
    f1 = jnp.sum(ent1 * wn, axis=1, keepdims=True)
    g = f0 * cls1_wt_ref[0:1, :] + f1 * cls1_wt_ref[1:2, :]
    g = jnp.maximum(g + cls1_b_ref[...], 0.0)
    logits = jnp.dot(g, cls2_wt_ref[...], preferred_element_type=jnp.float32)
    logits_ref[...] = logits + cls2_b_ref[...]
    w_ref[...] = w
    f_ref[...] = jnp.concatenate([f0, f1], axis=1)


def kernel(x, fc1_w, fc1_b, fc2_w, fc2_b, cls1_w, cls1_b, cls2_w, cls2_b):
    B, C, H, W = x.shape
    BC = B * C
    G = BC // _CH
    xr = x.reshape(BC, H, W)

    stat_shape = jax.ShapeDtypeStruct((G, 1, _CH), jnp.float32)
    stat_spec = pl.BlockSpec((1, 1, _CH), lambda i: (i, 0, 0))
    sx, s0, s1, t0, t1 = pl.pallas_call(
        _stats_body,
        grid=(G,),
        in_specs=[pl.BlockSpec((_CH, H, W), lambda i: (i, 0, 0))],
        out_specs=[stat_spec] * 5,
        out_shape=[stat_shape] * 5,
        compiler_params=pltpu.CompilerParams(
            dimension_semantics=("parallel",),
            vmem_limit_bytes=56 * 1024 * 1024,
        ),
    )(xr)

    def as_bc(v):
        return v.reshape(B, C)

    inv_hw = jnp.full((1, 1), 1.0 / (H * W), jnp.float32)
    logits, w, f = pl.pallas_call(
        _head_body,
        out_shape=[
            jax.ShapeDtypeStruct((B, cls2_w.shape[0]), jnp.float32),
            jax.ShapeDtypeStruct((B, C), jnp.float32),
            jax.ShapeDtypeStruct((B, 2), jnp.float32),
        ],
    )(as_bc(sx), as_bc(s0), as_bc(s1), as_bc(t0), as_bc(t1),
      fc1_w.T, fc1_b.reshape(1, -1), fc2_w.T, fc2_b.reshape(1, -1),
      cls1_w.T, cls1_b.reshape(1, -1), cls2_w.T, cls2_b.reshape(1, -1),
      inv_hw)
    return (logits, w, f)
